# Initial kernel scaffold; baseline (speedup 1.0000x reference)
import jax
import jax.numpy as jnp
from jax.experimental import pallas as pl


def kernel(p, x, o, wq, bq, wk, bk, wv, bv, wp1, bp1, gp, betap, wp2, bp2, g1, be1, w1, b1, g2, be2, w2, b2):
    raise NotImplementedError("write your pallas kernel here")


# speedup vs baseline: 5.0612x; 5.0612x over previous
import functools

import jax
import jax.numpy as jnp
from jax import lax
from jax.experimental import pallas as pl
from jax.experimental.pallas import tpu as pltpu
from jax.experimental.pallas import tpu_sc as plsc

N = 8192
NB = 4
NP = 2048
C = 256
MID = 256
CS = 32
NS = 16
K = N * NS
EPS = 1e-5


_RQ = 1024


def _qkv_body(x_ref, wq_ref, bq_ref, wk_ref, bk_ref, wv_ref, bv_ref,
              q_ref, k_ref, v_ref):
    xb = x_ref[...]
    q_ref[...] = jnp.dot(xb, wq_ref[...], preferred_element_type=jnp.float32) + bq_ref[...]
    k_ref[...] = jnp.dot(xb, wk_ref[...], preferred_element_type=jnp.float32) + bk_ref[...]
    v_ref[...] = jnp.dot(xb, wv_ref[...], preferred_element_type=jnp.float32) + bv_ref[...]


def _qkv(x, wq, bq, wk, bk, wv, bv):
    grid = (N // _RQ,)
    row = pl.BlockSpec((_RQ, C), lambda i: (i, 0))
    wspec = pl.BlockSpec((C, C), lambda i: (0, 0))
    bspec = pl.BlockSpec((1, C), lambda i: (0, 0))
    return pl.pallas_call(
        _qkv_body,
        grid=grid,
        in_specs=[row, wspec, bspec, wspec, bspec, wspec, bspec],
        out_specs=[row, row, row],
        out_shape=[jax.ShapeDtypeStruct((N, C), jnp.float32)] * 3,
    )(x, wq, bq[None], wk, bk[None], wv, bv[None])


_RK = 256


def _knn_body(p_ref, pT_ref, idx_ref):
    b = pl.program_id(0)
    pb = p_ref[...]
    pT = pT_ref[...]
    sqm = jnp.sum(pT * pT, axis=0, keepdims=True)
    sqr = jnp.sum(pb * pb, axis=1, keepdims=True)
    dot = jnp.dot(pb.astype(jnp.bfloat16), pT.astype(jnp.bfloat16),
                  preferred_element_type=jnp.float32)
    dist = sqr + sqm - 2.0 * dot
    iota = lax.broadcasted_iota(jnp.int32, (_RK, NP), 1)
    big = jnp.int32(0x7FFFFFFF)
    off = b * NP
    for t in range(NS):
        mn = jnp.min(dist, axis=1, keepdims=True)
        cand = jnp.where(dist == mn, iota, big)
        am = jnp.min(cand, axis=1, keepdims=True)
        idx_ref[:, t:t + 1] = am + off
        dist = jnp.where(cand == am, jnp.inf, dist)


def _knn(p, pT):
    tpb = NP // _RK
    grid = (NB, tpb)
    return pl.pallas_call(
        _knn_body,
        grid=grid,
        in_specs=[
            pl.BlockSpec((_RK, 3), lambda b, t: (b * tpb + t, 0)),
            pl.BlockSpec((3, NP), lambda b, t: (0, b)),
        ],
        out_specs=pl.BlockSpec((_RK, NS), lambda b, t: (b * tpb + t, 0)),
        out_shape=jax.ShapeDtypeStruct((N, NS), jnp.int32),
    )(p, pT)


def _gather_rows(table, idx_flat, chunk, dout=None):
    V, D = table.shape
    dout = D if dout is None else dout
    btot = idx_flat.shape[0]
    nw = 32
    bpw = btot // nw
    nchunks = bpw // chunk
    mesh = plsc.VectorSubcoreMesh(core_axis_name="c", subcore_axis_name="s")

    @functools.partial(
        pl.kernel,
        out_type=jax.ShapeDtypeStruct((btot, dout), table.dtype),
        mesh=mesh,
        scratch_types=[
            pltpu.VMEM((bpw,), jnp.int32),
            pltpu.VMEM((chunk, D), table.dtype),
            pltpu.SemaphoreType.DMA,
        ],
    )
    def k(table_hbm, idx_hbm, out_hbm, idx_v, rows_v, sem):
        wid = lax.axis_index("s") * 2 + lax.axis_index("c")
        base = wid * bpw
        pltpu.sync_copy(idx_hbm.at[pl.ds(base, bpw)], idx_v)

        @pl.loop(0, nchunks)
        def _(ci):
            pltpu.async_copy(
                table_hbm.at[idx_v.at[pl.ds(ci * chunk, chunk)]], rows_v, sem
            ).wait()
            src = rows_v if dout == D else rows_v.at[:, pl.ds(0, dout)]
            pltpu.sync_copy(src, out_hbm.at[pl.ds(base + ci * chunk, chunk)])

    return k(table, idx_flat)


_RC = 512


def _relstats_body(pg_ref, pp_ref, rel_ref, m1_ref, m2_ref):
    i = pl.program_id(0)
    rel = pg_ref[...][:, :, :16] - pp_ref[...][None]
    rel_ref[...] = rel

    @pl.when(i == 0)
    def _():
        m1_ref[...] = jnp.zeros_like(m1_ref)
        m2_ref[...] = jnp.zeros_like(m2_ref)

    m1_ref[...] += jnp.sum(rel, axis=(0, 1))[None]
    rows = [jnp.sum(rel * rel[:, :, d:d + 1], axis=(0, 1))[None]
            for d in range(3)]
    rows.append(jnp.zeros((5, 16), jnp.float32))
    m2_ref[...] += jnp.concatenate(rows, axis=0)


def _relstats(pg, p_pad):
    grid = (N // _RC,)
    return pl.pallas_call(
        _relstats_body,
        grid=grid,
        in_specs=[
            pl.BlockSpec((NS, _RC, 128), lambda i: (0, i, 0)),
            pl.BlockSpec((_RC, 16), lambda i: (i, 0)),
        ],
        out_specs=[pl.BlockSpec((NS, _RC, 16), lambda i: (0, i, 0)),
                   pl.BlockSpec((1, 16), lambda i: (0, 0)),
                   pl.BlockSpec((8, 16), lambda i: (0, 0))],
        out_shape=[jax.ShapeDtypeStruct((NS, N, 16), jnp.float32),
                   jax.ShapeDtypeStruct((1, 16), jnp.float32),
                   jax.ShapeDtypeStruct((8, 16), jnp.float32)],
        compiler_params=pltpu.CompilerParams(
            dimension_semantics=("arbitrary",)),
    )(pg, p_pad)


_RH = 512


def _h3_body(rel_ref, w_ref, b_ref, h3_ref):
    h = jnp.dot(rel_ref[...].reshape(NS * _RH, 16), w_ref[...],
                preferred_element_type=jnp.float32) + b_ref[...]
    h3_ref[...] = jnp.maximum(h, 0.0).reshape(NS, _RH, 16)


def _h3pass(relpad, wp1f, bp1f):
    grid = (N // _RH,)
    return pl.pallas_call(
        _h3_body,
        grid=grid,
        in_specs=[
            pl.BlockSpec((NS, _RH, 16), lambda i: (0, i, 0)),
            pl.BlockSpec((16, 16), lambda i: (0, 0)),
            pl.BlockSpec((1, 16), lambda i: (0, 0)),
        ],
        out_specs=pl.BlockSpec((NS, _RH, 16), lambda i: (0, i, 0)),
        out_shape=jax.ShapeDtypeStruct((NS, N, 16), jnp.float32),
    )(relpad, wp1f, bp1f)


_RE = 512


def _stats_body(xk_ref, xq_ref, h3_ref, wp2_ref, bp2_ref, s1_ref, s2_ref,
                acc1, acc2):
    i = pl.program_id(0)

    @pl.when(i == 0)
    def _():
        acc1[...] = jnp.zeros_like(acc1)
        acc2[...] = jnp.zeros_like(acc2)

    h3 = h3_ref[...].reshape(NS * _RE, 16)
    pr = jnp.dot(h3, wp2_ref[...], preferred_element_type=jnp.float32) + bp2_ref[...]
    r3 = xk_ref[...] - xq_ref[...][None] + pr.reshape(NS, _RE, C)
    acc1[...] += jnp.sum(r3, axis=(0, 1))[None]
    acc2[...] += jnp.sum(r3 * r3, axis=(0, 1))[None]

    @pl.when(i == pl.num_programs(0) - 1)
    def _():
        s1_ref[...] = acc1[...]
        s2_ref[...] = acc2[...]


def _stats(xkg, x_q, h3, wp2f, bp2r):
    grid = (N // _RE,)
    return pl.pallas_call(
        _stats_body,
        grid=grid,
        in_specs=[
            pl.BlockSpec((NS, _RE, C), lambda i: (0, i, 0)),
            pl.BlockSpec((_RE, C), lambda i: (i, 0)),
            pl.BlockSpec((NS, _RE, 16), lambda i: (0, i, 0)),
            pl.BlockSpec((16, C), lambda i: (0, 0)),
            pl.BlockSpec((1, C), lambda i: (0, 0)),
        ],
        out_specs=[pl.BlockSpec((1, C), lambda i: (0, 0)),
                   pl.BlockSpec((1, C), lambda i: (0, 0))],
        out_shape=[jax.ShapeDtypeStruct((1, C), jnp.float32)] * 2,
        scratch_shapes=[pltpu.VMEM((1, C), jnp.float32),
                        pltpu.VMEM((1, C), jnp.float32)],
        compiler_params=pltpu.CompilerParams(
            dimension_semantics=("arbitrary",)),
    )(xkg, x_q, h3, wp2f, bp2r)


def _fpass_body(xk_ref, xq_ref, h3_ref, wp2_ref, bp2_ref, sc1_ref, sh1_ref,
                w1_ref, b1_ref, h1_ref, s1_ref, s2_ref, acc1, acc2):
    i = pl.program_id(0)

    @pl.when(i == 0)
    def _():
        acc1[...] = jnp.zeros_like(acc1)
        acc2[...] = jnp.zeros_like(acc2)

    h3 = h3_ref[...].reshape(NS * _RE, 16)
    pr = jnp.dot(h3, wp2_ref[...], preferred_element_type=jnp.float32) + bp2_ref[...]
    r3 = xk_ref[...] - xq_ref[...][None] + pr.reshape(NS, _RE, C)
    a = jnp.maximum(r3 * sc1_ref[...][None] + sh1_ref[...][None], 0.0)
    h1 = jnp.dot(a.reshape(NS * _RE, C), w1_ref[...],
                 preferred_element_type=jnp.float32) + b1_ref[...]
    h1_ref[...] = h1.reshape(NS, _RE, CS)
    acc1[...] += jnp.sum(h1, axis=0)[None]
    acc2[...] += jnp.sum(h1 * h1, axis=0)[None]

    @pl.when(i == pl.num_programs(0) - 1)
    def _():
        s1_ref[...] = acc1[...]
        s2_ref[...] = acc2[...]


def _fpass(xkg, x_q, h3, wp2f, bp2r, sc1, sh1, w1, b1r):
    grid = (N // _RE,)
    vec = pl.BlockSpec((1, C), lambda i: (0, 0))
    return pl.pallas_call(
        _fpass_body,
        grid=grid,
        in_specs=[
            pl.BlockSpec((NS, _RE, C), lambda i: (0, i, 0)),
            pl.BlockSpec((_RE, C), lambda i: (i, 0)),
            pl.BlockSpec((NS, _RE, 16), lambda i: (0, i, 0)),
            pl.BlockSpec((16, C), lambda i: (0, 0)),
            vec, vec, vec,
            pl.BlockSpec((C, CS), lambda i: (0, 0)),
            pl.BlockSpec((1, CS), lambda i: (0, 0)),
        ],
        out_specs=[pl.BlockSpec((NS, _RE, CS), lambda i: (0, i, 0)),
                   pl.BlockSpec((1, CS), lambda i: (0, 0)),
                   pl.BlockSpec((1, CS), lambda i: (0, 0))],
        out_shape=[jax.ShapeDtypeStruct((NS, N, CS), jnp.float32),
                   jax.ShapeDtypeStruct((1, CS), jnp.float32),
                   jax.ShapeDtypeStruct((1, CS), jnp.float32)],
        scratch_shapes=[pltpu.VMEM((1, CS), jnp.float32),
                        pltpu.VMEM((1, CS), jnp.float32)],
        compiler_params=pltpu.CompilerParams(
            dimension_semantics=("arbitrary",)),
    )(xkg, x_q, h3, wp2f, bp2r, sc1, sh1, w1, b1r)


def _gpass_body(xv_ref, h3_ref, h1_ref, wp2_ref, bp2_ref, sc2_ref, sh2_ref,
                w2_ref, b2_ref, out_ref):
    h3 = h3_ref[...].reshape(NS * _RE, 16)
    pr = jnp.dot(h3, wp2_ref[...], preferred_element_type=jnp.float32) + bp2_ref[...]
    v3 = xv_ref[...] + pr.reshape(NS, _RE, C)
    a = jnp.maximum(h1_ref[...] * sc2_ref[...][None] + sh2_ref[...][None], 0.0)
    h2 = jnp.dot(a.reshape(NS * _RE, CS), w2_ref[...],
                 preferred_element_type=jnp.float32) + b2_ref[...]
    h2 = h2.reshape(NS, _RE, CS)
    m = jnp.max(h2, axis=0, keepdims=True)
    e = jnp.exp(h2 - m)
    w = e / jnp.sum(e, axis=0, keepdims=True)
    wf = jnp.concatenate([w] * 8, axis=2)
    out_ref[...] = jnp.sum(v3 * wf, axis=0)


def _gpass(xvg, h3, h1, wp2f, bp2r, sc2, sh2, w2, b2r):
    grid = (N // _RE,)
    vec32 = pl.BlockSpec((1, CS), lambda i: (0, 0))
    return pl.pallas_call(
        _gpass_body,
        grid=grid,
        in_specs=[
            pl.BlockSpec((NS, _RE, C), lambda i: (0, i, 0)),
            pl.BlockSpec((NS, _RE, 16), lambda i: (0, i, 0)),
            pl.BlockSpec((NS, _RE, CS), lambda i: (0, i, 0)),
            pl.BlockSpec((16, C), lambda i: (0, 0)),
            pl.BlockSpec((1, C), lambda i: (0, 0)),
            vec32, vec32,
            pl.BlockSpec((CS, CS), lambda i: (0, 0)),
            vec32,
        ],
        out_specs=pl.BlockSpec((_RE, C), lambda i: (i, 0)),
        out_shape=jax.ShapeDtypeStruct((N, C), jnp.float32),
    )(xvg, h3, h1, wp2f, bp2r, sc2, sh2, w2, b2r)


def kernel(p, x, o, wq, bq, wk, bk, wv, bv, wp1, bp1, gp, betap, wp2, bp2,
           g1, be1, w1, b1, g2, be2, w2, b2):
    f32 = jnp.float32
    x_q, x_k, x_v = _qkv(x, wq, bq, wk, bk, wv, bv)

    pT = jnp.transpose(p)
    idx = _knn(p, pT)
    idx_tm = jnp.transpose(idx).reshape(K)

    p_pad = jnp.pad(p, ((0, 0), (0, 13)))
    p_pad128 = jnp.pad(p, ((0, 0), (0, 125)))
    pg = _gather_rows(p_pad128, idx_tm, 512).reshape(NS, N, 128)

    relpad, m1, m2 = _relstats(pg, p_pad)
    mu = m1[0, :3] / K
    cov = m2[:3, :3] / K - jnp.outer(mu, mu)
    mh = mu @ wp1 + bp1
    vh = jnp.sum(wp1 * (cov @ wp1), axis=0)
    s = gp / jnp.sqrt(vh + EPS)
    wp1p = wp1 * s[None, :]
    bp1p = (bp1 - mh) * s + betap
    wp1f = jnp.zeros((16, 16), f32).at[:3, :3].set(wp1p)
    bp1f = jnp.zeros((1, 16), f32).at[0, :3].set(bp1p)

    h3 = _h3pass(relpad, wp1f, bp1f)
    wp2f = jnp.zeros((16, C), f32).at[:3, :].set(wp2)
    bp2r = bp2[None]

    xkg = _gather_rows(x_k, idx_tm, 128).reshape(NS, N, C)
    s1, s2 = _stats(xkg, x_q, h3, wp2f, bp2r)
    mean1 = s1[0] / K
    var1 = s2[0] / K - mean1 * mean1
    sc1 = (g1 / jnp.sqrt(var1 + EPS))[None]
    sh1 = (be1 - mean1 * sc1[0])[None]

    h1, t1, t2 = _fpass(xkg, x_q, h3, wp2f, bp2r, sc1, sh1, w1, b1[None])
    mean2 = t1[0] / K
    var2 = t2[0] / K - mean2 * mean2
    sc2 = (g2 / jnp.sqrt(var2 + EPS))[None]
    sh2 = (be2 - mean2 * sc2[0])[None]

    xvg = _gather_rows(x_v, idx_tm, 128).reshape(NS, N, C)
    out = _gpass(xvg, h3, h1, wp2f, bp2r, sc2, sh2, w2, b2[None])
    return out


# speedup vs baseline: 5.0794x; 1.0036x over previous
import functools

import jax
import jax.numpy as jnp
from jax import lax
from jax.experimental import pallas as pl
from jax.experimental.pallas import tpu as pltpu
from jax.experimental.pallas import tpu_sc as plsc

N = 8192
NB = 4
NP = 2048
C = 256
MID = 256
CS = 32
NS = 16
K = N * NS
EPS = 1e-5


_RQ = 1024


def _qkv_body(x_ref, wq_ref, bq_ref, wk_ref, bk_ref, wv_ref, bv_ref,
              q_ref, k_ref, v_ref):
    xb = x_ref[...]
    q_ref[...] = jnp.dot(xb, wq_ref[...], preferred_element_type=jnp.float32) + bq_ref[...]
    k_ref[...] = jnp.dot(xb, wk_ref[...], preferred_element_type=jnp.float32) + bk_ref[...]
    v_ref[...] = jnp.dot(xb, wv_ref[...], preferred_element_type=jnp.float32) + bv_ref[...]


def _qkv(x, wq, bq, wk, bk, wv, bv):
    grid = (N // _RQ,)
    row = pl.BlockSpec((_RQ, C), lambda i: (i, 0))
    wspec = pl.BlockSpec((C, C), lambda i: (0, 0))
    bspec = pl.BlockSpec((1, C), lambda i: (0, 0))
    return pl.pallas_call(
        _qkv_body,
        grid=grid,
        in_specs=[row, wspec, bspec, wspec, bspec, wspec, bspec],
        out_specs=[row, row, row],
        out_shape=[jax.ShapeDtypeStruct((N, C), jnp.float32)] * 3,
    )(x, wq, bq[None], wk, bk[None], wv, bv[None])


_RK = 256


def _knn_body(p_ref, pT_ref, idx_ref):
    b = pl.program_id(0)
    pb = p_ref[...]
    pT = pT_ref[...]
    sqm = jnp.sum(pT * pT, axis=0, keepdims=True)
    sqr = jnp.sum(pb * pb, axis=1, keepdims=True)
    dot = jnp.dot(pb.astype(jnp.bfloat16), pT.astype(jnp.bfloat16),
                  preferred_element_type=jnp.float32)
    dist = sqr + sqm - 2.0 * dot
    iota = lax.broadcasted_iota(jnp.int32, (_RK, NP), 1)
    big = jnp.int32(0x7FFFFFFF)
    off = b * NP
    for t in range(NS):
        mn = jnp.min(dist, axis=1, keepdims=True)
        cand = jnp.where(dist == mn, iota, big)
        am = jnp.min(cand, axis=1, keepdims=True)
        idx_ref[:, t:t + 1] = am + off
        dist = jnp.where(cand == am, jnp.inf, dist)


def _knn(p, pT):
    tpb = NP // _RK
    grid = (NB, tpb)
    return pl.pallas_call(
        _knn_body,
        grid=grid,
        in_specs=[
            pl.BlockSpec((_RK, 3), lambda b, t: (b * tpb + t, 0)),
            pl.BlockSpec((3, NP), lambda b, t: (0, b)),
        ],
        out_specs=pl.BlockSpec((_RK, NS), lambda b, t: (b * tpb + t, 0)),
        out_shape=jax.ShapeDtypeStruct((N, NS), jnp.int32),
    )(p, pT)


def _gather_rows(table, idx_flat, chunk, dout=None):
    V, D = table.shape
    dout = D if dout is None else dout
    btot = idx_flat.shape[0]
    nw = 32
    bpw = btot // nw
    nchunks = bpw // chunk
    mesh = plsc.VectorSubcoreMesh(core_axis_name="c", subcore_axis_name="s")

    assert nchunks % 2 == 0

    @functools.partial(
        pl.kernel,
        out_type=jax.ShapeDtypeStruct((btot, dout), table.dtype),
        mesh=mesh,
        scratch_types=[
            pltpu.VMEM((bpw,), jnp.int32),
            pltpu.VMEM((chunk, D), table.dtype),
            pltpu.VMEM((chunk, D), table.dtype),
            pltpu.SemaphoreType.DMA,
            pltpu.SemaphoreType.DMA,
            pltpu.SemaphoreType.DMA,
            pltpu.SemaphoreType.DMA,
        ],
    )
    def k(table_hbm, idx_hbm, out_hbm, idx_v, rows0, rows1, g0, g1, w0, w1):
        wid = lax.axis_index("s") * 2 + lax.axis_index("c")
        base = wid * bpw
        pltpu.sync_copy(idx_hbm.at[pl.ds(base, bpw)], idx_v)

        @pl.loop(0, nchunks, step=2)
        def _(ci):
            for buf, gsem, wsem, step in ((rows0, g0, w0, 0), (rows1, g1, w1, 1)):
                @pl.when(ci >= 2)
                def _():
                    pltpu.make_async_copy(buf, out_hbm.at[pl.ds(base, chunk)],
                                          wsem).wait()
                cc = ci + step
                pltpu.async_copy(
                    table_hbm.at[idx_v.at[pl.ds(cc * chunk, chunk)]], buf, gsem
                ).wait()
                pltpu.async_copy(buf, out_hbm.at[pl.ds(base + cc * chunk, chunk)],
                                 wsem)

        for buf, wsem in ((rows0, w0), (rows1, w1)):
            pltpu.make_async_copy(buf, out_hbm.at[pl.ds(base, chunk)], wsem).wait()

    return k(table, idx_flat)


_RC = 512


def _relstats_body(pg_ref, pp_ref, rel_ref, m1_ref, m2_ref):
    i = pl.program_id(0)
    rel = pg_ref[...][:, :, :16] - pp_ref[...][None]
    rel_ref[...] = rel

    @pl.when(i == 0)
    def _():
        m1_ref[...] = jnp.zeros_like(m1_ref)
        m2_ref[...] = jnp.zeros_like(m2_ref)

    m1_ref[...] += jnp.sum(rel, axis=(0, 1))[None]
    rows = [jnp.sum(rel * rel[:, :, d:d + 1], axis=(0, 1))[None]
            for d in range(3)]
    rows.append(jnp.zeros((5, 16), jnp.float32))
    m2_ref[...] += jnp.concatenate(rows, axis=0)


def _relstats(pg, p_pad):
    grid = (N // _RC,)
    return pl.pallas_call(
        _relstats_body,
        grid=grid,
        in_specs=[
            pl.BlockSpec((NS, _RC, 128), lambda i: (0, i, 0)),
            pl.BlockSpec((_RC, 16), lambda i: (i, 0)),
        ],
        out_specs=[pl.BlockSpec((NS, _RC, 16), lambda i: (0, i, 0)),
                   pl.BlockSpec((1, 16), lambda i: (0, 0)),
                   pl.BlockSpec((8, 16), lambda i: (0, 0))],
        out_shape=[jax.ShapeDtypeStruct((NS, N, 16), jnp.float32),
                   jax.ShapeDtypeStruct((1, 16), jnp.float32),
                   jax.ShapeDtypeStruct((8, 16), jnp.float32)],
        compiler_params=pltpu.CompilerParams(
            dimension_semantics=("arbitrary",)),
    )(pg, p_pad)


_RH = 512


def _h3_body(rel_ref, w_ref, b_ref, h3_ref):
    h = jnp.dot(rel_ref[...].reshape(NS * _RH, 16), w_ref[...],
                preferred_element_type=jnp.float32) + b_ref[...]
    h3_ref[...] = jnp.maximum(h, 0.0).reshape(NS, _RH, 16)


def _h3pass(relpad, wp1f, bp1f):
    grid = (N // _RH,)
    return pl.pallas_call(
        _h3_body,
        grid=grid,
        in_specs=[
            pl.BlockSpec((NS, _RH, 16), lambda i: (0, i, 0)),
            pl.BlockSpec((16, 16), lambda i: (0, 0)),
            pl.BlockSpec((1, 16), lambda i: (0, 0)),
        ],
        out_specs=pl.BlockSpec((NS, _RH, 16), lambda i: (0, i, 0)),
        out_shape=jax.ShapeDtypeStruct((NS, N, 16), jnp.float32),
    )(relpad, wp1f, bp1f)


_RE = 512


def _stats_body(xk_ref, xq_ref, h3_ref, wp2_ref, bp2_ref, s1_ref, s2_ref,
                acc1, acc2):
    i = pl.program_id(0)

    @pl.when(i == 0)
    def _():
        acc1[...] = jnp.zeros_like(acc1)
        acc2[...] = jnp.zeros_like(acc2)

    h3 = h3_ref[...].reshape(NS * _RE, 16)
    pr = jnp.dot(h3, wp2_ref[...], preferred_element_type=jnp.float32) + bp2_ref[...]
    r3 = xk_ref[...] - xq_ref[...][None] + pr.reshape(NS, _RE, C)
    acc1[...] += jnp.sum(r3, axis=(0, 1))[None]
    acc2[...] += jnp.sum(r3 * r3, axis=(0, 1))[None]

    @pl.when(i == pl.num_programs(0) - 1)
    def _():
        s1_ref[...] = acc1[...]
        s2_ref[...] = acc2[...]


def _stats(xkg, x_q, h3, wp2f, bp2r):
    grid = (N // _RE,)
    return pl.pallas_call(
        _stats_body,
        grid=grid,
        in_specs=[
            pl.BlockSpec((NS, _RE, C), lambda i: (0, i, 0)),
            pl.BlockSpec((_RE, C), lambda i: (i, 0)),
            pl.BlockSpec((NS, _RE, 16), lambda i: (0, i, 0)),
            pl.BlockSpec((16, C), lambda i: (0, 0)),
            pl.BlockSpec((1, C), lambda i: (0, 0)),
        ],
        out_specs=[pl.BlockSpec((1, C), lambda i: (0, 0)),
                   pl.BlockSpec((1, C), lambda i: (0, 0))],
        out_shape=[jax.ShapeDtypeStruct((1, C), jnp.float32)] * 2,
        scratch_shapes=[pltpu.VMEM((1, C), jnp.float32),
                        pltpu.VMEM((1, C), jnp.float32)],
        compiler_params=pltpu.CompilerParams(
            dimension_semantics=("arbitrary",)),
    )(xkg, x_q, h3, wp2f, bp2r)


def _fpass_body(xk_ref, xq_ref, h3_ref, wp2_ref, bp2_ref, sc1_ref, sh1_ref,
                w1_ref, b1_ref, h1_ref, s1_ref, s2_ref, acc1, acc2):
    i = pl.program_id(0)

    @pl.when(i == 0)
    def _():
        acc1[...] = jnp.zeros_like(acc1)
        acc2[...] = jnp.zeros_like(acc2)

    h3 = h3_ref[...].reshape(NS * _RE, 16)
    pr = jnp.dot(h3, wp2_ref[...], preferred_element_type=jnp.float32) + bp2_ref[...]
    r3 = xk_ref[...] - xq_ref[...][None] + pr.reshape(NS, _RE, C)
    a = jnp.maximum(r3 * sc1_ref[...][None] + sh1_ref[...][None], 0.0)
    h1 = jnp.dot(a.reshape(NS * _RE, C), w1_ref[...],
                 preferred_element_type=jnp.float32) + b1_ref[...]
    h1_ref[...] = h1.reshape(NS, _RE, CS)
    acc1[...] += jnp.sum(h1, axis=0)[None]
    acc2[...] += jnp.sum(h1 * h1, axis=0)[None]

    @pl.when(i == pl.num_programs(0) - 1)
    def _():
        s1_ref[...] = acc1[...]
        s2_ref[...] = acc2[...]


def _fpass(xkg, x_q, h3, wp2f, bp2r, sc1, sh1, w1, b1r):
    grid = (N // _RE,)
    vec = pl.BlockSpec((1, C), lambda i: (0, 0))
    return pl.pallas_call(
        _fpass_body,
        grid=grid,
        in_specs=[
            pl.BlockSpec((NS, _RE, C), lambda i: (0, i, 0)),
            pl.BlockSpec((_RE, C), lambda i: (i, 0)),
            pl.BlockSpec((NS, _RE, 16), lambda i: (0, i, 0)),
            pl.BlockSpec((16, C), lambda i: (0, 0)),
            vec, vec, vec,
            pl.BlockSpec((C, CS), lambda i: (0, 0)),
            pl.BlockSpec((1, CS), lambda i: (0, 0)),
        ],
        out_specs=[pl.BlockSpec((NS, _RE, CS), lambda i: (0, i, 0)),
                   pl.BlockSpec((1, CS), lambda i: (0, 0)),
                   pl.BlockSpec((1, CS), lambda i: (0, 0))],
        out_shape=[jax.ShapeDtypeStruct((NS, N, CS), jnp.float32),
                   jax.ShapeDtypeStruct((1, CS), jnp.float32),
                   jax.ShapeDtypeStruct((1, CS), jnp.float32)],
        scratch_shapes=[pltpu.VMEM((1, CS), jnp.float32),
                        pltpu.VMEM((1, CS), jnp.float32)],
        compiler_params=pltpu.CompilerParams(
            dimension_semantics=("arbitrary",)),
    )(xkg, x_q, h3, wp2f, bp2r, sc1, sh1, w1, b1r)


def _gpass_body(xv_ref, h3_ref, h1_ref, wp2_ref, bp2_ref, sc2_ref, sh2_ref,
                w2_ref, b2_ref, out_ref):
    h3 = h3_ref[...].reshape(NS * _RE, 16)
    pr = jnp.dot(h3, wp2_ref[...], preferred_element_type=jnp.float32) + bp2_ref[...]
    v3 = xv_ref[...] + pr.reshape(NS, _RE, C)
    a = jnp.maximum(h1_ref[...] * sc2_ref[...][None] + sh2_ref[...][None], 0.0)
    h2 = jnp.dot(a.reshape(NS * _RE, CS), w2_ref[...],
                 preferred_element_type=jnp.float32) + b2_ref[...]
    h2 = h2.reshape(NS, _RE, CS)
    m = jnp.max(h2, axis=0, keepdims=True)
    e = jnp.exp(h2 - m)
    w = e / jnp.sum(e, axis=0, keepdims=True)
    wf = jnp.concatenate([w] * 8, axis=2)
    out_ref[...] = jnp.sum(v3 * wf, axis=0)


def _gpass(xvg, h3, h1, wp2f, bp2r, sc2, sh2, w2, b2r):
    grid = (N // _RE,)
    vec32 = pl.BlockSpec((1, CS), lambda i: (0, 0))
    return pl.pallas_call(
        _gpass_body,
        grid=grid,
        in_specs=[
            pl.BlockSpec((NS, _RE, C), lambda i: (0, i, 0)),
            pl.BlockSpec((NS, _RE, 16), lambda i: (0, i, 0)),
            pl.BlockSpec((NS, _RE, CS), lambda i: (0, i, 0)),
            pl.BlockSpec((16, C), lambda i: (0, 0)),
            pl.BlockSpec((1, C), lambda i: (0, 0)),
            vec32, vec32,
            pl.BlockSpec((CS, CS), lambda i: (0, 0)),
            vec32,
        ],
        out_specs=pl.BlockSpec((_RE, C), lambda i: (i, 0)),
        out_shape=jax.ShapeDtypeStruct((N, C), jnp.float32),
    )(xvg, h3, h1, wp2f, bp2r, sc2, sh2, w2, b2r)


def kernel(p, x, o, wq, bq, wk, bk, wv, bv, wp1, bp1, gp, betap, wp2, bp2,
           g1, be1, w1, b1, g2, be2, w2, b2):
    f32 = jnp.float32
    x_q, x_k, x_v = _qkv(x, wq, bq, wk, bk, wv, bv)

    pT = jnp.transpose(p)
    idx = _knn(p, pT)
    idx_tm = jnp.transpose(idx).reshape(K)

    p_pad = jnp.pad(p, ((0, 0), (0, 13)))
    p_pad128 = jnp.pad(p, ((0, 0), (0, 125)))
    pg = _gather_rows(p_pad128, idx_tm, 256).reshape(NS, N, 128)

    relpad, m1, m2 = _relstats(pg, p_pad)
    mu = m1[0, :3] / K
    cov = m2[:3, :3] / K - jnp.outer(mu, mu)
    mh = mu @ wp1 + bp1
    vh = jnp.sum(wp1 * (cov @ wp1), axis=0)
    s = gp / jnp.sqrt(vh + EPS)
    wp1p = wp1 * s[None, :]
    bp1p = (bp1 - mh) * s + betap
    wp1f = jnp.zeros((16, 16), f32).at[:3, :3].set(wp1p)
    bp1f = jnp.zeros((1, 16), f32).at[0, :3].set(bp1p)

    h3 = _h3pass(relpad, wp1f, bp1f)
    wp2f = jnp.zeros((16, C), f32).at[:3, :].set(wp2)
    bp2r = bp2[None]

    xkg = _gather_rows(x_k, idx_tm, 128).reshape(NS, N, C)
    s1, s2 = _stats(xkg, x_q, h3, wp2f, bp2r)
    mean1 = s1[0] / K
    var1 = s2[0] / K - mean1 * mean1
    sc1 = (g1 / jnp.sqrt(var1 + EPS))[None]
    sh1 = (be1 - mean1 * sc1[0])[None]

    h1, t1, t2 = _fpass(xkg, x_q, h3, wp2f, bp2r, sc1, sh1, w1, b1[None])
    mean2 = t1[0] / K
    var2 = t2[0] / K - mean2 * mean2
    sc2 = (g2 / jnp.sqrt(var2 + EPS))[None]
    sh2 = (be2 - mean2 * sc2[0])[None]

    xvg = _gather_rows(x_v, idx_tm, 128).reshape(NS, N, C)
    out = _gpass(xvg, h3, h1, wp2f, bp2r, sc2, sh2, w2, b2[None])
    return out


# speedup vs baseline: 5.5613x; 1.0949x over previous
import functools

import jax
import jax.numpy as jnp
from jax import lax
from jax.experimental import pallas as pl
from jax.experimental.pallas import tpu as pltpu
from jax.experimental.pallas import tpu_sc as plsc

N = 8192
NB = 4
NP = 2048
C = 256
MID = 256
CS = 32
NS = 16
K = N * NS
EPS = 1e-5


_RQ = 1024


def _qkv_body(x_ref, wq_ref, bq_ref, wk_ref, bk_ref, wv_ref, bv_ref,
              q_ref, k_ref, v_ref):
    xb = x_ref[...]
    q_ref[...] = jnp.dot(xb, wq_ref[...], preferred_element_type=jnp.float32) + bq_ref[...]
    k_ref[...] = jnp.dot(xb, wk_ref[...], preferred_element_type=jnp.float32) + bk_ref[...]
    v_ref[...] = jnp.dot(xb, wv_ref[...], preferred_element_type=jnp.float32) + bv_ref[...]


def _qkv(x, wq, bq, wk, bk, wv, bv):
    grid = (N // _RQ,)
    row = pl.BlockSpec((_RQ, C), lambda i: (i, 0))
    wspec = pl.BlockSpec((C, C), lambda i: (0, 0))
    bspec = pl.BlockSpec((1, C), lambda i: (0, 0))
    return pl.pallas_call(
        _qkv_body,
        grid=grid,
        in_specs=[row, wspec, bspec, wspec, bspec, wspec, bspec],
        out_specs=[row, row, row],
        out_shape=[jax.ShapeDtypeStruct((N, C), jnp.float32)] * 3,
    )(x, wq, bq[None], wk, bk[None], wv, bv[None])


_RK = 256


def _knn_body(p_ref, pT_ref, idx_ref):
    b = pl.program_id(0)
    pb = p_ref[...]
    pT = pT_ref[...]
    sqm = jnp.sum(pT * pT, axis=0, keepdims=True)
    sqr = jnp.sum(pb * pb, axis=1, keepdims=True)
    dot = jnp.dot(pb.astype(jnp.bfloat16), pT.astype(jnp.bfloat16),
                  preferred_element_type=jnp.float32)
    dist = sqr + sqm - 2.0 * dot
    iota = lax.broadcasted_iota(jnp.int32, (_RK, NP), 1)
    off = b * NP
    for t in range(NS):
        am = jnp.argmin(dist, axis=1).astype(jnp.int32)[:, None]
        idx_ref[:, t:t + 1] = am + off
        dist = jnp.where(iota == am, jnp.inf, dist)


def _knn(p, pT):
    tpb = NP // _RK
    grid = (NB, tpb)
    return pl.pallas_call(
        _knn_body,
        grid=grid,
        in_specs=[
            pl.BlockSpec((_RK, 3), lambda b, t: (b * tpb + t, 0)),
            pl.BlockSpec((3, NP), lambda b, t: (0, b)),
        ],
        out_specs=pl.BlockSpec((_RK, NS), lambda b, t: (b * tpb + t, 0)),
        out_shape=jax.ShapeDtypeStruct((N, NS), jnp.int32),
    )(p, pT)


def _gather_rows(table, idx_flat, chunk, dout=None):
    V, D = table.shape
    dout = D if dout is None else dout
    btot = idx_flat.shape[0]
    nw = 32
    bpw = btot // nw
    nchunks = bpw // chunk
    mesh = plsc.VectorSubcoreMesh(core_axis_name="c", subcore_axis_name="s")

    assert nchunks % 2 == 0

    @functools.partial(
        pl.kernel,
        out_type=jax.ShapeDtypeStruct((btot, dout), table.dtype),
        mesh=mesh,
        scratch_types=[
            pltpu.VMEM((bpw,), jnp.int32),
            pltpu.VMEM((chunk, D), table.dtype),
            pltpu.VMEM((chunk, D), table.dtype),
            pltpu.SemaphoreType.DMA,
            pltpu.SemaphoreType.DMA,
            pltpu.SemaphoreType.DMA,
            pltpu.SemaphoreType.DMA,
        ],
    )
    def k(table_hbm, idx_hbm, out_hbm, idx_v, rows0, rows1, g0, g1, w0, w1):
        wid = lax.axis_index("s") * 2 + lax.axis_index("c")
        base = wid * bpw
        pltpu.sync_copy(idx_hbm.at[pl.ds(base, bpw)], idx_v)

        @pl.loop(0, nchunks, step=2)
        def _(ci):
            for buf, gsem, wsem, step in ((rows0, g0, w0, 0), (rows1, g1, w1, 1)):
                @pl.when(ci >= 2)
                def _():
                    pltpu.make_async_copy(buf, out_hbm.at[pl.ds(base, chunk)],
                                          wsem).wait()
                cc = ci + step
                pltpu.async_copy(
                    table_hbm.at[idx_v.at[pl.ds(cc * chunk, chunk)]], buf, gsem
                ).wait()
                pltpu.async_copy(buf, out_hbm.at[pl.ds(base + cc * chunk, chunk)],
                                 wsem)

        for buf, wsem in ((rows0, w0), (rows1, w1)):
            pltpu.make_async_copy(buf, out_hbm.at[pl.ds(base, chunk)], wsem).wait()

    return k(table, idx_flat)


_RC = 512


def _relstats_body(pg_ref, pp_ref, rel_ref, m1_ref, m2_ref):
    i = pl.program_id(0)
    rel = pg_ref[...][:, :, :16] - pp_ref[...][None]
    rel_ref[...] = rel

    @pl.when(i == 0)
    def _():
        m1_ref[...] = jnp.zeros_like(m1_ref)
        m2_ref[...] = jnp.zeros_like(m2_ref)

    m1_ref[...] += jnp.sum(rel, axis=(0, 1))[None]
    rows = [jnp.sum(rel * rel[:, :, d:d + 1], axis=(0, 1))[None]
            for d in range(3)]
    rows.append(jnp.zeros((5, 16), jnp.float32))
    m2_ref[...] += jnp.concatenate(rows, axis=0)


def _relstats(pg, p_pad):
    grid = (N // _RC,)
    return pl.pallas_call(
        _relstats_body,
        grid=grid,
        in_specs=[
            pl.BlockSpec((NS, _RC, 128), lambda i: (0, i, 0)),
            pl.BlockSpec((_RC, 16), lambda i: (i, 0)),
        ],
        out_specs=[pl.BlockSpec((NS, _RC, 16), lambda i: (0, i, 0)),
                   pl.BlockSpec((1, 16), lambda i: (0, 0)),
                   pl.BlockSpec((8, 16), lambda i: (0, 0))],
        out_shape=[jax.ShapeDtypeStruct((NS, N, 16), jnp.float32),
                   jax.ShapeDtypeStruct((1, 16), jnp.float32),
                   jax.ShapeDtypeStruct((8, 16), jnp.float32)],
        compiler_params=pltpu.CompilerParams(
            dimension_semantics=("arbitrary",)),
    )(pg, p_pad)


_RH = 512


def _h3_body(rel_ref, w_ref, b_ref, h3_ref):
    h = jnp.dot(rel_ref[...].reshape(NS * _RH, 16), w_ref[...],
                preferred_element_type=jnp.float32) + b_ref[...]
    h3_ref[...] = jnp.maximum(h, 0.0).reshape(NS, _RH, 16)


def _h3pass(relpad, wp1f, bp1f):
    grid = (N // _RH,)
    return pl.pallas_call(
        _h3_body,
        grid=grid,
        in_specs=[
            pl.BlockSpec((NS, _RH, 16), lambda i: (0, i, 0)),
            pl.BlockSpec((16, 16), lambda i: (0, 0)),
            pl.BlockSpec((1, 16), lambda i: (0, 0)),
        ],
        out_specs=pl.BlockSpec((NS, _RH, 16), lambda i: (0, i, 0)),
        out_shape=jax.ShapeDtypeStruct((NS, N, 16), jnp.float32),
    )(relpad, wp1f, bp1f)


_RE = 512


def _stats_body(xk_ref, xq_ref, h3_ref, wp2_ref, bp2_ref, s1_ref, s2_ref,
                acc1, acc2):
    i = pl.program_id(0)

    @pl.when(i == 0)
    def _():
        acc1[...] = jnp.zeros_like(acc1)
        acc2[...] = jnp.zeros_like(acc2)

    h3 = h3_ref[...].reshape(NS * _RE, 16)
    pr = jnp.dot(h3, wp2_ref[...], preferred_element_type=jnp.float32) + bp2_ref[...]
    r3 = xk_ref[...] - xq_ref[...][None] + pr.reshape(NS, _RE, C)
    acc1[...] += jnp.sum(r3, axis=(0, 1))[None]
    acc2[...] += jnp.sum(r3 * r3, axis=(0, 1))[None]

    @pl.when(i == pl.num_programs(0) - 1)
    def _():
        s1_ref[...] = acc1[...]
        s2_ref[...] = acc2[...]


def _stats(xkg, x_q, h3, wp2f, bp2r):
    grid = (N // _RE,)
    return pl.pallas_call(
        _stats_body,
        grid=grid,
        in_specs=[
            pl.BlockSpec((NS, _RE, C), lambda i: (0, i, 0)),
            pl.BlockSpec((_RE, C), lambda i: (i, 0)),
            pl.BlockSpec((NS, _RE, 16), lambda i: (0, i, 0)),
            pl.BlockSpec((16, C), lambda i: (0, 0)),
            pl.BlockSpec((1, C), lambda i: (0, 0)),
        ],
        out_specs=[pl.BlockSpec((1, C), lambda i: (0, 0)),
                   pl.BlockSpec((1, C), lambda i: (0, 0))],
        out_shape=[jax.ShapeDtypeStruct((1, C), jnp.float32)] * 2,
        scratch_shapes=[pltpu.VMEM((1, C), jnp.float32),
                        pltpu.VMEM((1, C), jnp.float32)],
        compiler_params=pltpu.CompilerParams(
            dimension_semantics=("arbitrary",)),
    )(xkg, x_q, h3, wp2f, bp2r)


def _fpass_body(xk_ref, xq_ref, h3_ref, wp2_ref, bp2_ref, sc1_ref, sh1_ref,
                w1_ref, b1_ref, h1_ref, s1_ref, s2_ref, acc1, acc2):
    i = pl.program_id(0)

    @pl.when(i == 0)
    def _():
        acc1[...] = jnp.zeros_like(acc1)
        acc2[...] = jnp.zeros_like(acc2)

    h3 = h3_ref[...].reshape(NS * _RE, 16)
    pr = jnp.dot(h3, wp2_ref[...], preferred_element_type=jnp.float32) + bp2_ref[...]
    r3 = xk_ref[...] - xq_ref[...][None] + pr.reshape(NS, _RE, C)
    a = jnp.maximum(r3 * sc1_ref[...][None] + sh1_ref[...][None], 0.0)
    h1 = jnp.dot(a.reshape(NS * _RE, C), w1_ref[...],
                 preferred_element_type=jnp.float32) + b1_ref[...]
    h1_ref[...] = h1.reshape(NS, _RE, CS)
    acc1[...] += jnp.sum(h1, axis=0)[None]
    acc2[...] += jnp.sum(h1 * h1, axis=0)[None]

    @pl.when(i == pl.num_programs(0) - 1)
    def _():
        s1_ref[...] = acc1[...]
        s2_ref[...] = acc2[...]


def _fpass(xkg, x_q, h3, wp2f, bp2r, sc1, sh1, w1, b1r):
    grid = (N // _RE,)
    vec = pl.BlockSpec((1, C), lambda i: (0, 0))
    return pl.pallas_call(
        _fpass_body,
        grid=grid,
        in_specs=[
            pl.BlockSpec((NS, _RE, C), lambda i: (0, i, 0)),
            pl.BlockSpec((_RE, C), lambda i: (i, 0)),
            pl.BlockSpec((NS, _RE, 16), lambda i: (0, i, 0)),
            pl.BlockSpec((16, C), lambda i: (0, 0)),
            vec, vec, vec,
            pl.BlockSpec((C, CS), lambda i: (0, 0)),
            pl.BlockSpec((1, CS), lambda i: (0, 0)),
        ],
        out_specs=[pl.BlockSpec((NS, _RE, CS), lambda i: (0, i, 0)),
                   pl.BlockSpec((1, CS), lambda i: (0, 0)),
                   pl.BlockSpec((1, CS), lambda i: (0, 0))],
        out_shape=[jax.ShapeDtypeStruct((NS, N, CS), jnp.float32),
                   jax.ShapeDtypeStruct((1, CS), jnp.float32),
                   jax.ShapeDtypeStruct((1, CS), jnp.float32)],
        scratch_shapes=[pltpu.VMEM((1, CS), jnp.float32),
                        pltpu.VMEM((1, CS), jnp.float32)],
        compiler_params=pltpu.CompilerParams(
            dimension_semantics=("arbitrary",)),
    )(xkg, x_q, h3, wp2f, bp2r, sc1, sh1, w1, b1r)


def _gpass_body(xv_ref, h3_ref, h1_ref, wp2_ref, bp2_ref, sc2_ref, sh2_ref,
                w2_ref, b2_ref, out_ref):
    h3 = h3_ref[...].reshape(NS * _RE, 16)
    pr = jnp.dot(h3, wp2_ref[...], preferred_element_type=jnp.float32) + bp2_ref[...]
    v3 = xv_ref[...] + pr.reshape(NS, _RE, C)
    a = jnp.maximum(h1_ref[...] * sc2_ref[...][None] + sh2_ref[...][None], 0.0)
    h2 = jnp.dot(a.reshape(NS * _RE, CS), w2_ref[...],
                 preferred_element_type=jnp.float32) + b2_ref[...]
    h2 = h2.reshape(NS, _RE, CS)
    m = jnp.max(h2, axis=0, keepdims=True)
    e = jnp.exp(h2 - m)
    w = e / jnp.sum(e, axis=0, keepdims=True)
    wf = jnp.concatenate([w] * 8, axis=2)
    out_ref[...] = jnp.sum(v3 * wf, axis=0)


def _gpass(xvg, h3, h1, wp2f, bp2r, sc2, sh2, w2, b2r):
    grid = (N // _RE,)
    vec32 = pl.BlockSpec((1, CS), lambda i: (0, 0))
    return pl.pallas_call(
        _gpass_body,
        grid=grid,
        in_specs=[
            pl.BlockSpec((NS, _RE, C), lambda i: (0, i, 0)),
            pl.BlockSpec((NS, _RE, 16), lambda i: (0, i, 0)),
            pl.BlockSpec((NS, _RE, CS), lambda i: (0, i, 0)),
            pl.BlockSpec((16, C), lambda i: (0, 0)),
            pl.BlockSpec((1, C), lambda i: (0, 0)),
            vec32, vec32,
            pl.BlockSpec((CS, CS), lambda i: (0, 0)),
            vec32,
        ],
        out_specs=pl.BlockSpec((_RE, C), lambda i: (i, 0)),
        out_shape=jax.ShapeDtypeStruct((N, C), jnp.float32),
    )(xvg, h3, h1, wp2f, bp2r, sc2, sh2, w2, b2r)


def kernel(p, x, o, wq, bq, wk, bk, wv, bv, wp1, bp1, gp, betap, wp2, bp2,
           g1, be1, w1, b1, g2, be2, w2, b2):
    f32 = jnp.float32
    x_q, x_k, x_v = _qkv(x, wq, bq, wk, bk, wv, bv)

    pT = jnp.transpose(p)
    idx = _knn(p, pT)
    idx_tm = jnp.transpose(idx).reshape(K)

    p_pad = jnp.pad(p, ((0, 0), (0, 13)))
    p_pad128 = jnp.pad(p, ((0, 0), (0, 125)))
    pg = _gather_rows(p_pad128, idx_tm, 256).reshape(NS, N, 128)

    relpad, m1, m2 = _relstats(pg, p_pad)
    mu = m1[0, :3] / K
    cov = m2[:3, :3] / K - jnp.outer(mu, mu)
    mh = mu @ wp1 + bp1
    vh = jnp.sum(wp1 * (cov @ wp1), axis=0)
    s = gp / jnp.sqrt(vh + EPS)
    wp1p = wp1 * s[None, :]
    bp1p = (bp1 - mh) * s + betap
    wp1f = jnp.zeros((16, 16), f32).at[:3, :3].set(wp1p)
    bp1f = jnp.zeros((1, 16), f32).at[0, :3].set(bp1p)

    h3 = _h3pass(relpad, wp1f, bp1f)
    wp2f = jnp.zeros((16, C), f32).at[:3, :].set(wp2)
    bp2r = bp2[None]

    xkg = _gather_rows(x_k, idx_tm, 128).reshape(NS, N, C)
    s1, s2 = _stats(xkg, x_q, h3, wp2f, bp2r)
    mean1 = s1[0] / K
    var1 = s2[0] / K - mean1 * mean1
    sc1 = (g1 / jnp.sqrt(var1 + EPS))[None]
    sh1 = (be1 - mean1 * sc1[0])[None]

    h1, t1, t2 = _fpass(xkg, x_q, h3, wp2f, bp2r, sc1, sh1, w1, b1[None])
    mean2 = t1[0] / K
    var2 = t2[0] / K - mean2 * mean2
    sc2 = (g2 / jnp.sqrt(var2 + EPS))[None]
    sh2 = (be2 - mean2 * sc2[0])[None]

    xvg = _gather_rows(x_v, idx_tm, 128).reshape(NS, N, C)
    out = _gpass(xvg, h3, h1, wp2f, bp2r, sc2, sh2, w2, b2[None])
    return out
